# baseline (speedup 1.0000x reference)
import functools

import jax
import jax.numpy as jnp
from jax import lax
from jax.experimental import pallas as pl
from jax.experimental.pallas import tpu as pltpu
from jax.experimental.pallas import tpu_sc as plsc

N = 10000
E = 160000
D = 256

NC = 2
NS = 16
LANES = 16
NCHUNK = 4
CW = D // NCHUNK
SW = 2 * CW
EPW = E // NS
BATCH = 80
RUNROLL = 2
NPIPE = 2
NB = EPW // BATCH
NP = 10240
NRANGE = 2
RSPAN = NP // NRANGE
DUMMY = 64
ACCROWS = RSPAN + DUMMY
RPS = RSPAN // NS
FLUSH = 64
NFLUSH = RPS // FLUSH
ZROWS = 32


def _project_kernel(h_ref, w_ref, b_ref, s_ref, bb_ref):
    hb = h_ref[...]
    p = jnp.dot(hb, w_ref[...], preferred_element_type=jnp.float32)
    parts = []
    for c in range(NCHUNK):
        parts.append(p[:, CW * c:CW * (c + 1)])
        parts.append(hb[:, CW * c:CW * (c + 1)])
    s_ref[...] = jnp.concatenate(parts, axis=1)
    bb_ref[...] = p[:, D:] + b_ref[...]


def _node_projections(h, wab, b2):
    blk = 2000
    grid = N // blk
    return pl.pallas_call(
        _project_kernel,
        grid=(grid,),
        in_specs=[
            pl.BlockSpec((blk, D), lambda i: (i, 0)),
            pl.BlockSpec((D, 2 * D), lambda i: (0, 0)),
            pl.BlockSpec((1, D), lambda i: (0, 0)),
        ],
        out_specs=[
            pl.BlockSpec((blk, 2 * D), lambda i: (i, 0)),
            pl.BlockSpec((blk, D), lambda i: (i, 0)),
        ],
        out_shape=[
            jax.ShapeDtypeStruct((N, 2 * D), jnp.float32),
            jax.ShapeDtypeStruct((N, D), jnp.float32),
        ],
    )(h, wab, b2)


def _edge_body(stbl, btbl, eidx, out,
               sraw, draw,
               gs0, gs1, gd0, gd1, gl0, gl1,
               sbuf0, sbuf1, dbuf0, dbuf1, obuf0, obuf1,
               zbuf, acc,
               sem_s0, sem_s1, sem_d0, sem_d1, sem_o0, sem_o1):
    cid = lax.axis_index("c")
    sid = lax.axis_index("s")

    pltpu.sync_copy(eidx.at[0, sid], sraw)
    pltpu.sync_copy(eidx.at[1, sid], draw)

    zero16 = jnp.zeros((LANES,), jnp.float32)

    def zrow(r, carry):
        for j in range(2 * CW // LANES):
            zbuf[r, pl.ds(LANES * j, LANES)] = zero16
        return carry

    lax.fori_loop(0, ZROWS, zrow, 0)

    sets = ((sbuf0, dbuf0, obuf0, sem_s0, sem_d0, sem_o0, gs0, gd0, gl0),
            (sbuf1, dbuf1, obuf1, sem_s1, sem_d1, sem_o1, gs1, gd1, gl1))

    def issue(bi, chunk, s):
        gs, gd = s[6], s[7]
        for j in range(BATCH // LANES):
            sl = pl.ds(LANES * j, LANES)
            gs[sl] = sraw[bi, sl] * NCHUNK + chunk
            gd[sl] = draw[bi, sl] * NCHUNK + chunk
        pltpu.async_copy(stbl.at[gs], s[0], s[3])
        pltpu.async_copy(btbl.at[gd], s[1], s[4])

    def set_scatter_idx(bi, rbase, s):
        gl = s[8]
        for j in range(BATCH // LANES):
            sl = pl.ds(LANES * j, LANES)
            ld = draw[bi, sl] - rbase
            ok = (ld >= 0) & (ld < RSPAN)
            gl[sl] = jnp.where(ok, ld, RSPAN + (ld & (DUMMY - 1)))

    def wait_gathers(s):
        pltpu.make_async_copy(stbl.at[s[6]], s[0], s[3]).wait()
        pltpu.make_async_copy(btbl.at[s[7]], s[1], s[4]).wait()

    def compute(s):
        sb, db, ob = s[0], s[1], s[2]

        def rowblk(r0, c2):
            base = r0 * RUNROLL
            for dr in range(RUNROLL):
                r = base + dr
                for j in range(CW // LANES):
                    sl = pl.ds(LANES * j, LANES)
                    slh = pl.ds(CW + LANES * j, LANES)
                    x = sb[r, sl] + db[r, sl]
                    w = jnp.exp(jnp.maximum(x, x * 0.01))
                    ob[r, sl] = w
                    ob[r, slh] = sb[r, slh] * w
            return c2

        lax.fori_loop(0, BATCH // RUNROLL, rowblk, 0)

    def scatter(s):
        pltpu.async_copy(s[2], acc.at[s[8]], s[5], add=True)

    def wait_scatter(s):
        pltpu.make_async_copy(s[2], acc.at[s[8]], s[5]).wait()

    for kc in range(NCHUNK // NC):
        chunk = cid * (NCHUNK // NC) + kc
        for rg in range(NRANGE):
            rbase = rg * RSPAN

            for z in range(RPS // ZROWS):
                pltpu.sync_copy(zbuf, acc.at[pl.ds(sid * RPS + z * ZROWS, ZROWS)])
            plsc.subcore_barrier()

            for ph in range(NPIPE):
                issue(ph, chunk, sets[ph])

            def step(bp, carry):
                for ph in range(NPIPE):
                    s = sets[ph]
                    bi = NPIPE * bp + ph

                    @pl.when(bp > 0)
                    def _():
                        wait_scatter(s)

                    wait_gathers(s)
                    compute(s)
                    set_scatter_idx(bi, rbase, s)
                    scatter(s)
                    issue(jnp.minimum(bi + NPIPE, NB - 1), chunk, s)
                return carry

            lax.fori_loop(0, (NB - 1) // NPIPE, step, 0)

            wait_scatter(sets[0])
            wait_gathers(sets[0])
            compute(sets[0])
            set_scatter_idx(NB - 1, rbase, sets[0])
            scatter(sets[0])
            for ph in range(1, NPIPE):
                wait_gathers(sets[ph])
                wait_scatter(sets[ph])
            wait_scatter(sets[0])
            plsc.subcore_barrier()

            fbuf = sbuf0
            wbuf = dbuf0
            for z in range(NFLUSH):
                row0 = sid * RPS + z * FLUSH
                pltpu.sync_copy(acc.at[pl.ds(row0, FLUSH)], fbuf.at[pl.ds(0, FLUSH)])

                def drow(r0, carry):
                    for dr in range(RUNROLL):
                        r = r0 * RUNROLL + dr
                        for j in range(CW // LANES):
                            sl = pl.ds(LANES * j, LANES)
                            den = fbuf[r, sl]
                            num = fbuf[r, pl.ds(CW + LANES * j, LANES)]
                            wbuf[r, sl] = jnp.where(den > 0.0, num / den, 0.0)
                    return carry

                lax.fori_loop(0, FLUSH // RUNROLL, drow, 0)
                pltpu.sync_copy(wbuf.at[pl.ds(0, FLUSH)], out.at[chunk, pl.ds(rbase + row0, FLUSH)])


_edge_kernel = functools.partial(
    pl.kernel,
    out_type=jax.ShapeDtypeStruct((NCHUNK, NP, CW), jnp.float32),
    mesh=plsc.VectorSubcoreMesh(
        core_axis_name="c", subcore_axis_name="s", num_cores=NC, num_subcores=NS
    ),
    scratch_types=[
        pltpu.VMEM((NB, BATCH), jnp.int32),
        pltpu.VMEM((NB, BATCH), jnp.int32),
        pltpu.VMEM((BATCH,), jnp.int32),
        pltpu.VMEM((BATCH,), jnp.int32),
        pltpu.VMEM((BATCH,), jnp.int32),
        pltpu.VMEM((BATCH,), jnp.int32),
        pltpu.VMEM((BATCH,), jnp.int32),
        pltpu.VMEM((BATCH,), jnp.int32),
        pltpu.VMEM((BATCH, SW), jnp.float32),
        pltpu.VMEM((BATCH, SW), jnp.float32),
        pltpu.VMEM((BATCH, CW), jnp.float32),
        pltpu.VMEM((BATCH, CW), jnp.float32),
        pltpu.VMEM((BATCH, 2 * CW), jnp.float32),
        pltpu.VMEM((BATCH, 2 * CW), jnp.float32),
        pltpu.VMEM((ZROWS, 2 * CW), jnp.float32),
        pltpu.VMEM_SHARED((ACCROWS, 2 * CW), jnp.float32),
    ] + [pltpu.SemaphoreType.DMA] * 6,
    compiler_params=pltpu.CompilerParams(use_tc_tiling_on_sc=False),
)(_edge_body)


def kernel(h, edge_index, W, b):
    wab = jnp.concatenate([W[:, :D].T, W[:, D:].T], axis=1)
    b2 = b.reshape(1, D)
    s, bb = _node_projections(h, wab, b2)
    stbl = s.reshape(NCHUNK * N, SW)
    btbl = bb.reshape(NCHUNK * N, CW)
    eidx = edge_index.reshape(2, NS, NB, BATCH)
    outc = _edge_kernel(stbl, btbl, eidx)
    return outc[:, :N].transpose(1, 0, 2).reshape(N, D)


# speedup vs baseline: 1.7672x; 1.7672x over previous
import functools

import jax
import jax.numpy as jnp
from jax import lax
from jax.experimental import pallas as pl
from jax.experimental.pallas import tpu as pltpu
from jax.experimental.pallas import tpu_sc as plsc

N = 10000
E = 160000
D = 256

NC = 2
NS = 16
LANES = 16
NCHUNK = 8
CW = D // NCHUNK
SW = 2 * CW
EPW = E // NS
BATCH = 80
RUNROLL = 2
NPIPE = 2
NB = EPW // BATCH
NP = 10240
NRANGE = 1
RSPAN = NP // NRANGE
DUMMY = 64
ACCROWS = RSPAN + DUMMY
RPS = RSPAN // NS
FLUSH = 64
NFLUSH = RPS // FLUSH
ZROWS = 32


def _project_kernel(h_ref, w_ref, b_ref, s_ref, bb_ref):
    hb = h_ref[...]
    p = jnp.dot(hb, w_ref[...], preferred_element_type=jnp.float32)
    parts = []
    for c in range(NCHUNK):
        parts.append(p[:, CW * c:CW * (c + 1)])
        parts.append(hb[:, CW * c:CW * (c + 1)])
    s_ref[...] = jnp.concatenate(parts, axis=1)
    bb_ref[...] = p[:, D:] + b_ref[...]


def _node_projections(h, wab, b2):
    blk = 2000
    grid = N // blk
    return pl.pallas_call(
        _project_kernel,
        grid=(grid,),
        in_specs=[
            pl.BlockSpec((blk, D), lambda i: (i, 0)),
            pl.BlockSpec((D, 2 * D), lambda i: (0, 0)),
            pl.BlockSpec((1, D), lambda i: (0, 0)),
        ],
        out_specs=[
            pl.BlockSpec((blk, 2 * D), lambda i: (i, 0)),
            pl.BlockSpec((blk, D), lambda i: (i, 0)),
        ],
        out_shape=[
            jax.ShapeDtypeStruct((N, 2 * D), jnp.float32),
            jax.ShapeDtypeStruct((N, D), jnp.float32),
        ],
    )(h, wab, b2)


def _edge_body(stbl, btbl, eidx, out,
               sraw, draw,
               gs0, gs1, gd0, gd1, gl0, gl1,
               sbuf0, sbuf1, dbuf0, dbuf1, obuf0, obuf1,
               zbuf, acc,
               sem_s0, sem_s1, sem_d0, sem_d1, sem_o0, sem_o1):
    cid = lax.axis_index("c")
    sid = lax.axis_index("s")

    pltpu.sync_copy(eidx.at[0, sid], sraw)
    pltpu.sync_copy(eidx.at[1, sid], draw)

    zero16 = jnp.zeros((LANES,), jnp.float32)

    def zrow(r, carry):
        for j in range(2 * CW // LANES):
            zbuf[r, pl.ds(LANES * j, LANES)] = zero16
        return carry

    lax.fori_loop(0, ZROWS, zrow, 0)

    sets = ((sbuf0, dbuf0, obuf0, sem_s0, sem_d0, sem_o0, gs0, gd0, gl0),
            (sbuf1, dbuf1, obuf1, sem_s1, sem_d1, sem_o1, gs1, gd1, gl1))

    def issue(bi, chunk, s):
        gs, gd = s[6], s[7]
        for j in range(BATCH // LANES):
            sl = pl.ds(LANES * j, LANES)
            gs[sl] = sraw[bi, sl] * NCHUNK + chunk
            gd[sl] = draw[bi, sl] * NCHUNK + chunk
        pltpu.async_copy(stbl.at[gs], s[0], s[3])
        pltpu.async_copy(btbl.at[gd], s[1], s[4])

    def set_scatter_idx(bi, rbase, s):
        gl = s[8]
        for j in range(BATCH // LANES):
            sl = pl.ds(LANES * j, LANES)
            ld = draw[bi, sl] - rbase
            ok = (ld >= 0) & (ld < RSPAN)
            gl[sl] = jnp.where(ok, ld, RSPAN + (ld & (DUMMY - 1)))

    def wait_gathers(s):
        pltpu.make_async_copy(stbl.at[s[6]], s[0], s[3]).wait()
        pltpu.make_async_copy(btbl.at[s[7]], s[1], s[4]).wait()

    def compute(s):
        sb, db, ob = s[0], s[1], s[2]

        def rowblk(r0, c2):
            base = r0 * RUNROLL
            for dr in range(RUNROLL):
                r = base + dr
                for j in range(CW // LANES):
                    sl = pl.ds(LANES * j, LANES)
                    slh = pl.ds(CW + LANES * j, LANES)
                    x = sb[r, sl] + db[r, sl]
                    w = jnp.exp(jnp.maximum(x, x * 0.01))
                    ob[r, sl] = w
                    ob[r, slh] = sb[r, slh] * w
            return c2

        lax.fori_loop(0, BATCH // RUNROLL, rowblk, 0)

    def scatter(s):
        pltpu.async_copy(s[2], acc.at[s[8]], s[5], add=True)

    def wait_scatter(s):
        pltpu.make_async_copy(s[2], acc.at[s[8]], s[5]).wait()

    for kc in range(NCHUNK // NC):
        chunk = cid * (NCHUNK // NC) + kc
        for rg in range(NRANGE):
            rbase = rg * RSPAN

            for z in range(RPS // ZROWS):
                pltpu.sync_copy(zbuf, acc.at[pl.ds(sid * RPS + z * ZROWS, ZROWS)])
            plsc.subcore_barrier()

            for ph in range(NPIPE):
                issue(ph, chunk, sets[ph])

            def step(bp, carry):
                for ph in range(NPIPE):
                    s = sets[ph]
                    bi = NPIPE * bp + ph

                    @pl.when(bp > 0)
                    def _():
                        wait_scatter(s)

                    wait_gathers(s)
                    compute(s)
                    set_scatter_idx(bi, rbase, s)
                    scatter(s)
                    issue(jnp.minimum(bi + NPIPE, NB - 1), chunk, s)
                return carry

            lax.fori_loop(0, (NB - 1) // NPIPE, step, 0)

            wait_scatter(sets[0])
            wait_gathers(sets[0])
            compute(sets[0])
            set_scatter_idx(NB - 1, rbase, sets[0])
            scatter(sets[0])
            for ph in range(1, NPIPE):
                wait_gathers(sets[ph])
                wait_scatter(sets[ph])
            wait_scatter(sets[0])
            plsc.subcore_barrier()

            fbuf = sbuf0
            wbuf = dbuf0
            for z in range(NFLUSH):
                row0 = sid * RPS + z * FLUSH
                pltpu.sync_copy(acc.at[pl.ds(row0, FLUSH)], fbuf.at[pl.ds(0, FLUSH)])

                def drow(r0, carry):
                    for dr in range(RUNROLL):
                        r = r0 * RUNROLL + dr
                        for j in range(CW // LANES):
                            sl = pl.ds(LANES * j, LANES)
                            den = fbuf[r, sl]
                            num = fbuf[r, pl.ds(CW + LANES * j, LANES)]
                            wbuf[r, sl] = jnp.where(den > 0.0, num / den, 0.0)
                    return carry

                lax.fori_loop(0, FLUSH // RUNROLL, drow, 0)
                pltpu.sync_copy(wbuf.at[pl.ds(0, FLUSH)], out.at[chunk, pl.ds(rbase + row0, FLUSH)])


_edge_kernel = functools.partial(
    pl.kernel,
    out_type=jax.ShapeDtypeStruct((NCHUNK, NP, CW), jnp.float32),
    mesh=plsc.VectorSubcoreMesh(
        core_axis_name="c", subcore_axis_name="s", num_cores=NC, num_subcores=NS
    ),
    scratch_types=[
        pltpu.VMEM((NB, BATCH), jnp.int32),
        pltpu.VMEM((NB, BATCH), jnp.int32),
        pltpu.VMEM((BATCH,), jnp.int32),
        pltpu.VMEM((BATCH,), jnp.int32),
        pltpu.VMEM((BATCH,), jnp.int32),
        pltpu.VMEM((BATCH,), jnp.int32),
        pltpu.VMEM((BATCH,), jnp.int32),
        pltpu.VMEM((BATCH,), jnp.int32),
        pltpu.VMEM((BATCH, SW), jnp.float32),
        pltpu.VMEM((BATCH, SW), jnp.float32),
        pltpu.VMEM((BATCH, CW), jnp.float32),
        pltpu.VMEM((BATCH, CW), jnp.float32),
        pltpu.VMEM((BATCH, 2 * CW), jnp.float32),
        pltpu.VMEM((BATCH, 2 * CW), jnp.float32),
        pltpu.VMEM((ZROWS, 2 * CW), jnp.float32),
        pltpu.VMEM_SHARED((ACCROWS, 2 * CW), jnp.float32),
    ] + [pltpu.SemaphoreType.DMA] * 6,
    compiler_params=pltpu.CompilerParams(use_tc_tiling_on_sc=False),
)(_edge_body)


def kernel(h, edge_index, W, b):
    wab = jnp.concatenate([W[:, :D].T, W[:, D:].T], axis=1)
    b2 = b.reshape(1, D)
    s, bb = _node_projections(h, wab, b2)
    stbl = s.reshape(NCHUNK * N, SW)
    btbl = bb.reshape(NCHUNK * N, CW)
    eidx = edge_index.reshape(2, NS, NB, BATCH)
    outc = _edge_kernel(stbl, btbl, eidx)
    return outc[:, :N].transpose(1, 0, 2).reshape(N, D)


# speedup vs baseline: 1.7980x; 1.0174x over previous
import functools

import jax
import jax.numpy as jnp
from jax import lax
from jax.experimental import pallas as pl
from jax.experimental.pallas import tpu as pltpu
from jax.experimental.pallas import tpu_sc as plsc

N = 10000
E = 160000
D = 256

NC = 2
NS = 16
LANES = 16
NCHUNK = 8
CW = D // NCHUNK
SW = 2 * CW
EPW = E // NS
BATCH = 80
RUNROLL = 2
NPIPE = 2
NB = EPW // BATCH
NP = 10240
NRANGE = 1
RSPAN = NP // NRANGE
DUMMY = 64
ACCROWS = RSPAN + DUMMY
RPS = RSPAN // NS
FLUSH = 64
NFLUSH = RPS // FLUSH
ZROWS = 32


def _project_kernel(h_ref, w_ref, b_ref, s_ref, bb_ref):
    hb = h_ref[...]
    p = jnp.dot(hb, w_ref[...], preferred_element_type=jnp.float32)
    parts = []
    for c in range(NCHUNK):
        parts.append(p[:, CW * c:CW * (c + 1)])
        parts.append(hb[:, CW * c:CW * (c + 1)])
    s_ref[...] = jnp.concatenate(parts, axis=1)
    bb_ref[...] = p[:, D:] + b_ref[...]


def _node_projections(h, wab, b2):
    blk = 2000
    grid = N // blk
    return pl.pallas_call(
        _project_kernel,
        grid=(grid,),
        in_specs=[
            pl.BlockSpec((blk, D), lambda i: (i, 0)),
            pl.BlockSpec((D, 2 * D), lambda i: (0, 0)),
            pl.BlockSpec((1, D), lambda i: (0, 0)),
        ],
        out_specs=[
            pl.BlockSpec((blk, 2 * D), lambda i: (i, 0)),
            pl.BlockSpec((blk, D), lambda i: (i, 0)),
        ],
        out_shape=[
            jax.ShapeDtypeStruct((N, 2 * D), jnp.float32),
            jax.ShapeDtypeStruct((N, D), jnp.float32),
        ],
    )(h, wab, b2)


def _edge_body(stbl, btbl, eidx, out,
               sraw, draw, sidx, didx,
               sbuf0, sbuf1, dbuf0, dbuf1, obuf0, obuf1,
               zbuf, acc,
               sem_s0, sem_s1, sem_d0, sem_d1, sem_o0, sem_o1):
    cid = lax.axis_index("c")
    sid = lax.axis_index("s")

    pltpu.sync_copy(eidx.at[0, sid], sraw)
    pltpu.sync_copy(eidx.at[1, sid], draw)

    zero16 = jnp.zeros((LANES,), jnp.float32)

    def zrow(r, carry):
        for j in range(2 * CW // LANES):
            zbuf[r, pl.ds(LANES * j, LANES)] = zero16
        return carry

    lax.fori_loop(0, ZROWS, zrow, 0)

    sets = ((sbuf0, dbuf0, obuf0, sem_s0, sem_d0, sem_o0),
            (sbuf1, dbuf1, obuf1, sem_s1, sem_d1, sem_o1))

    def issue(bi, s):
        pltpu.async_copy(stbl.at[sidx.at[bi]], s[0], s[3])
        pltpu.async_copy(btbl.at[didx.at[bi]], s[1], s[4])

    def wait_gathers(bi, s):
        pltpu.make_async_copy(stbl.at[sidx.at[bi]], s[0], s[3]).wait()
        pltpu.make_async_copy(btbl.at[didx.at[bi]], s[1], s[4]).wait()

    def compute(s):
        sb, db, ob = s[0], s[1], s[2]

        def rowblk(r0, c2):
            base = r0 * RUNROLL
            for dr in range(RUNROLL):
                r = base + dr
                for j in range(CW // LANES):
                    sl = pl.ds(LANES * j, LANES)
                    slh = pl.ds(CW + LANES * j, LANES)
                    x = sb[r, sl] + db[r, sl]
                    w = jnp.exp(jnp.maximum(x, x * 0.01))
                    ob[r, sl] = w
                    ob[r, slh] = sb[r, slh] * w
            return c2

        lax.fori_loop(0, BATCH // RUNROLL, rowblk, 0)

    def scatter(bi, s):
        pltpu.async_copy(s[2], acc.at[draw.at[bi]], s[5], add=True)

    def wait_scatter(bi, s):
        pltpu.make_async_copy(s[2], acc.at[draw.at[bi]], s[5]).wait()

    for kc in range(NCHUNK // NC):
        chunk = cid * (NCHUNK // NC) + kc
        for rg in range(NRANGE):
            rbase = rg * RSPAN

            def adj(r0, carry):
                for dr in range(5):
                    r = r0 * 5 + dr
                    for j in range(BATCH // LANES):
                        sl = pl.ds(LANES * j, LANES)
                        sidx[r, sl] = sraw[r, sl] * NCHUNK + chunk
                        didx[r, sl] = draw[r, sl] * NCHUNK + chunk
                return carry

            lax.fori_loop(0, NB // 5, adj, 0)

            for z in range(RPS // ZROWS):
                pltpu.sync_copy(zbuf, acc.at[pl.ds(sid * RPS + z * ZROWS, ZROWS)])
            plsc.subcore_barrier()

            for ph in range(NPIPE):
                issue(ph, sets[ph])

            def step(bp, carry):
                for ph in range(NPIPE):
                    s = sets[ph]
                    bi = NPIPE * bp + ph

                    @pl.when(bp > 0)
                    def _():
                        wait_scatter(bi - NPIPE, s)

                    wait_gathers(bi, s)
                    compute(s)
                    scatter(bi, s)
                    issue(jnp.minimum(bi + NPIPE, NB - 1), s)
                return carry

            lax.fori_loop(0, (NB - 1) // NPIPE, step, 0)

            wait_scatter(NB - 1 - NPIPE, sets[0])
            wait_gathers(NB - 1, sets[0])
            compute(sets[0])
            scatter(NB - 1, sets[0])
            for ph in range(1, NPIPE):
                wait_gathers(NB - 1, sets[ph])
                wait_scatter(NB - 1 - NPIPE + ph, sets[ph])
            wait_scatter(NB - 1, sets[0])
            plsc.subcore_barrier()

            fbuf = sbuf0
            wbuf = dbuf0
            for z in range(NFLUSH):
                row0 = sid * RPS + z * FLUSH
                pltpu.sync_copy(acc.at[pl.ds(row0, FLUSH)], fbuf.at[pl.ds(0, FLUSH)])

                def drow(r0, carry):
                    for dr in range(RUNROLL):
                        r = r0 * RUNROLL + dr
                        for j in range(CW // LANES):
                            sl = pl.ds(LANES * j, LANES)
                            den = fbuf[r, sl]
                            num = fbuf[r, pl.ds(CW + LANES * j, LANES)]
                            wbuf[r, sl] = jnp.where(den > 0.0, num / den, 0.0)
                    return carry

                lax.fori_loop(0, FLUSH // RUNROLL, drow, 0)
                pltpu.sync_copy(wbuf.at[pl.ds(0, FLUSH)], out.at[chunk, pl.ds(rbase + row0, FLUSH)])


_edge_kernel = functools.partial(
    pl.kernel,
    out_type=jax.ShapeDtypeStruct((NCHUNK, NP, CW), jnp.float32),
    mesh=plsc.VectorSubcoreMesh(
        core_axis_name="c", subcore_axis_name="s", num_cores=NC, num_subcores=NS
    ),
    scratch_types=[
        pltpu.VMEM((NB, BATCH), jnp.int32),
        pltpu.VMEM((NB, BATCH), jnp.int32),
        pltpu.VMEM((NB, BATCH), jnp.int32),
        pltpu.VMEM((NB, BATCH), jnp.int32),
        pltpu.VMEM((BATCH, SW), jnp.float32),
        pltpu.VMEM((BATCH, SW), jnp.float32),
        pltpu.VMEM((BATCH, CW), jnp.float32),
        pltpu.VMEM((BATCH, CW), jnp.float32),
        pltpu.VMEM((BATCH, 2 * CW), jnp.float32),
        pltpu.VMEM((BATCH, 2 * CW), jnp.float32),
        pltpu.VMEM((ZROWS, 2 * CW), jnp.float32),
        pltpu.VMEM_SHARED((ACCROWS, 2 * CW), jnp.float32),
    ] + [pltpu.SemaphoreType.DMA] * 6,
    compiler_params=pltpu.CompilerParams(use_tc_tiling_on_sc=False),
)(_edge_body)


def kernel(h, edge_index, W, b):
    wab = jnp.concatenate([W[:, :D].T, W[:, D:].T], axis=1)
    b2 = b.reshape(1, D)
    s, bb = _node_projections(h, wab, b2)
    stbl = s.reshape(NCHUNK * N, SW)
    btbl = bb.reshape(NCHUNK * N, CW)
    eidx = edge_index.reshape(2, NS, NB, BATCH)
    outc = _edge_kernel(stbl, btbl, eidx)
    return outc[:, :N].transpose(1, 0, 2).reshape(N, D)
